# baseline (speedup 1.0000x reference)
import jax
import jax.numpy as jnp
from jax.experimental import pallas as pl
from jax.experimental.pallas import tpu as pltpu

ROWS = 128
COLS = 100000
BLOCK_ROWS = 8
GROUPS = ROWS // BLOCK_ROWS
NBUF = 8

CHUNK = 12544
_BOUNDS = [(i * CHUNK, min((i + 1) * CHUNK, COLS)) for i in range(8)]


def _logsoftmax(x):
    chunks = [x[:, lo:hi] for lo, hi in _BOUNDS]
    partial_max = [jnp.max(c, axis=-1, keepdims=True) for c in chunks]
    m = partial_max[0]
    for pm in partial_max[1:]:
        m = jnp.maximum(m, pm)
    partial_sum = [jnp.sum(jnp.exp(c - m), axis=-1, keepdims=True) for c in chunks]
    s = partial_sum[0]
    for ps in partial_sum[1:]:
        s = s + ps
    return x - (m + jnp.log(s))


def _pipeline(x_hbm, o_hbm, xbuf, obuf, rsem, wsem):
    def read_copies(g):
        slot = g % NBUF
        return [
            pltpu.make_async_copy(
                x_hbm.at[g * BLOCK_ROWS + r, :],
                xbuf.at[slot, r],
                rsem.at[slot, r],
            )
            for r in range(BLOCK_ROWS)
        ]

    def write_copies(g):
        slot = g % NBUF
        return [
            pltpu.make_async_copy(
                obuf.at[slot, r],
                o_hbm.at[g * BLOCK_ROWS + r, :],
                wsem.at[slot, r],
            )
            for r in range(BLOCK_ROWS)
        ]

    for g in range(NBUF):
        for c in read_copies(g):
            c.start()

    for g in range(GROUPS):
        slot = g % NBUF
        for c in read_copies(g):
            c.wait()
        result = _logsoftmax(xbuf[slot])
        if g >= NBUF:
            for c in write_copies(g - NBUF):
                c.wait()
        obuf[slot] = result
        for c in write_copies(g):
            c.start()
        if g + NBUF < GROUPS:
            for c in read_copies(g + NBUF):
                c.start()

    for g in range(GROUPS - NBUF, GROUPS):
        for c in write_copies(g):
            c.wait()


def kernel(policy):
    return pl.pallas_call(
        _pipeline,
        in_specs=[pl.BlockSpec(memory_space=pltpu.MemorySpace.HBM)],
        out_specs=pl.BlockSpec(memory_space=pltpu.MemorySpace.HBM),
        out_shape=jax.ShapeDtypeStruct((ROWS, COLS), jnp.float32),
        scratch_shapes=[
            pltpu.VMEM((NBUF, BLOCK_ROWS, COLS), jnp.float32),
            pltpu.VMEM((NBUF, BLOCK_ROWS, COLS), jnp.float32),
            pltpu.SemaphoreType.DMA((NBUF, BLOCK_ROWS)),
            pltpu.SemaphoreType.DMA((NBUF, BLOCK_ROWS)),
        ],
    )(policy)


# speedup vs baseline: 1.0067x; 1.0067x over previous
import jax
import jax.numpy as jnp
from jax.experimental import pallas as pl
from jax.experimental.pallas import tpu as pltpu

ROWS = 128
COLS = 100000
BLOCK_ROWS = 8
GROUPS = ROWS // BLOCK_ROWS
NBUF = 8

CHUNK = 12544
_BOUNDS = [(i * CHUNK, min((i + 1) * CHUNK, COLS)) for i in range(8)]


def _logsoftmax(x):
    chunks = [x[:, lo:hi] for lo, hi in _BOUNDS]
    partial_max = [jnp.max(c, axis=-1, keepdims=True) for c in chunks]
    m = partial_max[0]
    for pm in partial_max[1:]:
        m = jnp.maximum(m, pm)
    partial_sum = [jnp.sum(jnp.exp(c - m), axis=-1, keepdims=True) for c in chunks]
    s = partial_sum[0]
    for ps in partial_sum[1:]:
        s = s + ps
    return x - (m + jnp.log(s))


def _pipeline(x_hbm, o_hbm, xbuf, obuf, rsem, wsem):
    def read_copy(g):
        slot = g % NBUF
        return pltpu.make_async_copy(
            x_hbm.at[pl.ds(g * BLOCK_ROWS, BLOCK_ROWS), :],
            xbuf.at[slot],
            rsem.at[slot],
        )

    def write_copy(g):
        slot = g % NBUF
        return pltpu.make_async_copy(
            obuf.at[slot],
            o_hbm.at[pl.ds(g * BLOCK_ROWS, BLOCK_ROWS), :],
            wsem.at[slot],
        )

    for g in range(NBUF):
        read_copy(g).start()

    for g in range(GROUPS):
        slot = g % NBUF
        read_copy(g).wait()
        result = _logsoftmax(xbuf[slot])
        if g >= NBUF:
            write_copy(g - NBUF).wait()
        obuf[slot] = result
        write_copy(g).start()
        if g + NBUF < GROUPS:
            read_copy(g + NBUF).start()

    for g in range(GROUPS - NBUF, GROUPS):
        write_copy(g).wait()


def kernel(policy):
    return pl.pallas_call(
        _pipeline,
        in_specs=[pl.BlockSpec(memory_space=pltpu.MemorySpace.HBM)],
        out_specs=pl.BlockSpec(memory_space=pltpu.MemorySpace.HBM),
        out_shape=jax.ShapeDtypeStruct((ROWS, COLS), jnp.float32),
        input_output_aliases={0: 0},
        scratch_shapes=[
            pltpu.VMEM((NBUF, BLOCK_ROWS, COLS), jnp.float32),
            pltpu.VMEM((NBUF, BLOCK_ROWS, COLS), jnp.float32),
            pltpu.SemaphoreType.DMA((NBUF,)),
            pltpu.SemaphoreType.DMA((NBUF,)),
        ],
    )(policy)


# speedup vs baseline: 1.0073x; 1.0007x over previous
import jax
import jax.numpy as jnp
from jax.experimental import pallas as pl
from jax.experimental.pallas import tpu as pltpu

ROWS = 128
COLS = 100000
BLOCK_ROWS = 8
GROUPS = ROWS // BLOCK_ROWS
NBUF = 4

CHUNK = 12544
_BOUNDS = [(i * CHUNK, min((i + 1) * CHUNK, COLS)) for i in range(8)]


def _logsoftmax(x):
    chunks = [x[:, lo:hi] for lo, hi in _BOUNDS]
    partial_max = [jnp.max(c, axis=-1, keepdims=True) for c in chunks]
    m = partial_max[0]
    for pm in partial_max[1:]:
        m = jnp.maximum(m, pm)
    partial_sum = [jnp.sum(jnp.exp(c - m), axis=-1, keepdims=True) for c in chunks]
    s = partial_sum[0]
    for ps in partial_sum[1:]:
        s = s + ps
    return x - (m + jnp.log(s))


def _pipeline(x_hbm, o_hbm, xbuf, obuf, rsem, wsem):
    def read_copy(g):
        slot = g % NBUF
        return pltpu.make_async_copy(
            x_hbm.at[pl.ds(g * BLOCK_ROWS, BLOCK_ROWS), :],
            xbuf.at[slot],
            rsem.at[slot],
        )

    def write_copy(g):
        slot = g % NBUF
        return pltpu.make_async_copy(
            obuf.at[slot],
            o_hbm.at[pl.ds(g * BLOCK_ROWS, BLOCK_ROWS), :],
            wsem.at[slot],
        )

    for g in range(NBUF):
        read_copy(g).start()

    for g in range(GROUPS):
        slot = g % NBUF
        read_copy(g).wait()
        result = _logsoftmax(xbuf[slot])
        if g >= NBUF:
            write_copy(g - NBUF).wait()
        obuf[slot] = result
        write_copy(g).start()
        if g + NBUF < GROUPS:
            read_copy(g + NBUF).start()

    for g in range(GROUPS - NBUF, GROUPS):
        write_copy(g).wait()


def kernel(policy):
    return pl.pallas_call(
        _pipeline,
        in_specs=[pl.BlockSpec(memory_space=pltpu.MemorySpace.HBM)],
        out_specs=pl.BlockSpec(memory_space=pltpu.MemorySpace.HBM),
        out_shape=jax.ShapeDtypeStruct((ROWS, COLS), jnp.float32),
        scratch_shapes=[
            pltpu.VMEM((NBUF, BLOCK_ROWS, COLS), jnp.float32),
            pltpu.VMEM((NBUF, BLOCK_ROWS, COLS), jnp.float32),
            pltpu.SemaphoreType.DMA((NBUF,)),
            pltpu.SemaphoreType.DMA((NBUF,)),
        ],
    )(policy)
